# Initial kernel scaffold; baseline (speedup 1.0000x reference)
import jax
import jax.numpy as jnp
from jax.experimental import pallas as pl


def kernel(atoms, edge_index, batch, params):
    raise NotImplementedError("write your pallas kernel here")


# speedup vs baseline: 2.7326x; 2.7326x over previous
import functools

import jax
import jax.numpy as jnp
from jax import lax
from jax.experimental import pallas as pl
from jax.experimental.pallas import tpu as pltpu
from jax.experimental.pallas import tpu_sc as plsc

F32 = jnp.float32
D = 128
NN = 10000
NE = 320000
NG = 256
NW = 32
C = 80
LN_EPS = 1e-5


def _sc_mesh():
    return plsc.VectorSubcoreMesh(core_axis_name="c", subcore_axis_name="s")


def _sc_gather_pair(hA, hB, dst, src):
    E = dst.shape[0]
    n_chunks = E // C
    per_w = n_chunks // NW

    @functools.partial(
        pl.kernel,
        out_type=(jax.ShapeDtypeStruct((E, D), F32),
                  jax.ShapeDtypeStruct((E, D), F32)),
        mesh=_sc_mesh(),
        scratch_types=[
            pltpu.VMEM((C,), jnp.int32),
            pltpu.VMEM((C,), jnp.int32),
            pltpu.VMEM((C, D), F32),
            pltpu.VMEM((C, D), F32),
            pltpu.SemaphoreType.DMA,
            pltpu.SemaphoreType.DMA,
        ],
    )
    def k(hA_hbm, hB_hbm, dst_hbm, src_hbm, gA_hbm, gB_hbm,
          iA, iB, bA, bB, semA, semB):
        wid = lax.axis_index("s") * 2 + lax.axis_index("c")

        def body(j, carry):
            base = pl.multiple_of((wid * per_w + j) * C, C)
            pltpu.sync_copy(dst_hbm.at[pl.ds(base, C)], iA)
            pltpu.sync_copy(src_hbm.at[pl.ds(base, C)], iB)
            cA = pltpu.async_copy(hA_hbm.at[iA], bA, semA)
            cB = pltpu.async_copy(hB_hbm.at[iB], bB, semB)
            cA.wait()
            cB.wait()
            pltpu.sync_copy(bA, gA_hbm.at[pl.ds(base, C)])
            pltpu.sync_copy(bB, gB_hbm.at[pl.ds(base, C)])
            return carry

        lax.fori_loop(0, per_w, body, 0)

    return k(hA, hB, dst, src)


def _sc_scatter_add(vals, idx, zeros):
    E = idx.shape[0]
    R = zeros.shape[0]
    n_chunks = E // C
    nj = -(-n_chunks // NW)
    rps = R // 16

    @functools.partial(
        pl.kernel,
        out_type=jax.ShapeDtypeStruct((2, R, D), F32),
        mesh=_sc_mesh(),
        scratch_types=[
            pltpu.VMEM((C,), jnp.int32),
            pltpu.VMEM((C, D), F32),
            pltpu.VMEM_SHARED((R, D), F32),
        ],
    )
    def k(vals_hbm, idx_hbm, zeros_hbm, out_hbm, ibuf, vbuf, acc):
        cid = lax.axis_index("c")
        sid = lax.axis_index("s")
        wid = sid * 2 + cid
        row0 = pl.multiple_of(sid * rps, 8)
        pltpu.sync_copy(zeros_hbm.at[pl.ds(row0, rps)], acc.at[pl.ds(row0, rps)])
        plsc.subcore_barrier()

        def body(j, carry):
            chunk = wid + NW * j

            @pl.when(chunk < n_chunks)
            def _():
                base = pl.multiple_of(chunk * C, C)
                pltpu.sync_copy(idx_hbm.at[pl.ds(base, C)], ibuf)
                pltpu.sync_copy(vals_hbm.at[pl.ds(base, C)], vbuf)
                pltpu.sync_copy(vbuf, acc.at[ibuf], add=True)

            return carry

        lax.fori_loop(0, nj, body, 0)
        plsc.subcore_barrier()
        pltpu.sync_copy(acc.at[pl.ds(row0, rps)],
                        out_hbm.at[cid, pl.ds(row0, rps)])

    return k(vals, idx, zeros)


def _ln_relu(x, g, b):
    mu = jnp.mean(x, axis=-1, keepdims=True)
    d = x - mu
    var = jnp.mean(d * d, axis=-1, keepdims=True)
    return jnp.maximum(d * lax.rsqrt(var + LN_EPS) * g + b, 0.0)


def _full(shape):
    return pl.BlockSpec(shape, lambda i: tuple(0 for _ in shape))


def _tc_embed_pre(atoms_f, emb, A, B, b1):
    Rb, grid = 1000, NN // 1000

    def body(a_ref, emb_ref, A_ref, B_ref, b_ref, h_ref, hA_ref, hB_ref):
        iota = lax.broadcasted_iota(jnp.int32, (Rb, 64), 1).astype(F32)
        oh = (a_ref[...] == iota).astype(F32)
        h = jnp.dot(oh, emb_ref[...], preferred_element_type=F32)
        h_ref[...] = h
        hA_ref[...] = jnp.dot(h, A_ref[...], preferred_element_type=F32) + b_ref[...]
        hB_ref[...] = jnp.dot(h, B_ref[...], preferred_element_type=F32)

    out = pl.pallas_call(
        body,
        grid=(grid,),
        in_specs=[pl.BlockSpec((Rb, 1), lambda i: (i, 0)),
                  _full((64, D)), _full((D, D)), _full((D, D)), _full((1, D))],
        out_specs=[pl.BlockSpec((Rb, D), lambda i: (i, 0))] * 3,
        out_shape=[jax.ShapeDtypeStruct((NN, D), F32)] * 3,
    )(atoms_f, emb, A, B, b1)
    return out


def _tc_edge_mlp(gA, gB, g1, B1, W2, b2, g2, B2):
    Rb = 2000
    grid = NE // Rb

    def body(gA_ref, gB_ref, g1r, B1r, W2r, b2r, g2r, B2r, m_ref):
        x = _ln_relu(gA_ref[...] + gB_ref[...], g1r[...], B1r[...])
        y = jnp.dot(x, W2r[...], preferred_element_type=F32) + b2r[...]
        m_ref[...] = _ln_relu(y, g2r[...], B2r[...])

    return pl.pallas_call(
        body,
        grid=(grid,),
        in_specs=[pl.BlockSpec((Rb, D), lambda i: (i, 0)),
                  pl.BlockSpec((Rb, D), lambda i: (i, 0)),
                  _full((1, D)), _full((1, D)), _full((D, D)),
                  _full((1, D)), _full((1, D)), _full((1, D))],
        out_specs=pl.BlockSpec((Rb, D), lambda i: (i, 0)),
        out_shape=jax.ShapeDtypeStruct((NE, D), F32),
    )(gA, gB, g1, B1, W2, b2, g2, B2)


def _tc_node_update(h, P, u, nxt):
    Rb, grid = 1000, NN // 1000
    uW1a, uW1b, ub1, ug1, uB1, uW2, ub2, ug2, uB2 = u

    def body(h_ref, P_ref, W1a, W1b, b1r, g1r, B1r, W2r, b2r, g2r, B2r,
             *rest):
        hv = h_ref[...]
        aggr = P_ref[0] + P_ref[1]
        x = (jnp.dot(hv, W1a[...], preferred_element_type=F32)
             + jnp.dot(aggr, W1b[...], preferred_element_type=F32) + b1r[...])
        x = _ln_relu(x, g1r[...], B1r[...])
        x = jnp.dot(x, W2r[...], preferred_element_type=F32) + b2r[...]
        hn = hv + _ln_relu(x, g2r[...], B2r[...])
        if nxt is None:
            (hn_ref,) = rest
            hn_ref[...] = hn
        else:
            An, Bn, bn, hn_ref, hAn_ref, hBn_ref = rest
            hn_ref[...] = hn
            hAn_ref[...] = jnp.dot(hn, An[...], preferred_element_type=F32) + bn[...]
            hBn_ref[...] = jnp.dot(hn, Bn[...], preferred_element_type=F32)

    w_specs = [_full((D, D)), _full((D, D)), _full((1, D)), _full((1, D)),
               _full((1, D)), _full((D, D)), _full((1, D)), _full((1, D)),
               _full((1, D))]
    in_specs = [pl.BlockSpec((Rb, D), lambda i: (i, 0)),
                pl.BlockSpec((2, Rb, D), lambda i: (0, i, 0))] + w_specs
    args = [h, P] + list(u)
    n_out = 1 if nxt is None else 3
    if nxt is not None:
        in_specs += [_full((D, D)), _full((D, D)), _full((1, D))]
        args += list(nxt)
    return pl.pallas_call(
        body,
        grid=(grid,),
        in_specs=in_specs,
        out_specs=[pl.BlockSpec((Rb, D), lambda i: (i, 0))] * n_out,
        out_shape=[jax.ShapeDtypeStruct((NN, D), F32)] * n_out,
    )(*args)


def _tc_head(Pp, W1, b1, W2p, b2p):
    def body(P_ref, W1r, b1r, W2r, b2r, out_ref):
        pooled = P_ref[0] + P_ref[1]
        hid = jnp.maximum(
            jnp.dot(pooled, W1r[...], preferred_element_type=F32) + b1r[...], 0.0)
        out_ref[...] = jnp.dot(hid, W2r[...], preferred_element_type=F32) + b2r[...]

    return pl.pallas_call(
        body,
        grid=(1,),
        in_specs=[_full((2, NG, D)), _full((D, D)), _full((1, D)),
                  _full((D, D)), _full((1, D))],
        out_specs=_full((NG, D)),
        out_shape=jax.ShapeDtypeStruct((NG, D), F32),
    )(Pp, W1, b1, W2p, b2p)


def kernel(atoms, edge_index, batch, params):
    src = edge_index[0]
    dst = edge_index[1]
    layers = params["layers"]
    atoms_f = atoms.astype(F32).reshape(NN, 1)
    NNp = -(-NN // 128) * 128
    z_nodes = jnp.zeros((NNp, D), F32)
    z_graphs = jnp.zeros((NG, D), F32)

    def row(v):
        return v.reshape(1, D)

    def pre_w(p):
        return p["mW1"][:D], p["mW1"][D:], row(p["mb1"])

    h, hA, hB = _tc_embed_pre(atoms_f, params["emb"], *pre_w(layers[0]))
    for li, p in enumerate(layers):
        gA, gB = _sc_gather_pair(hA, hB, dst, src)
        m = _tc_edge_mlp(gA, gB, row(p["mg1"]), row(p["mB1"]), p["mW2"],
                         row(p["mb2"]), row(p["mg2"]), row(p["mB2"]))
        P = _sc_scatter_add(m, dst, z_nodes)
        u = (p["uW1"][:D], p["uW1"][D:], row(p["ub1"]), row(p["ug1"]),
             row(p["uB1"]), p["uW2"], row(p["ub2"]), row(p["ug2"]),
             row(p["uB2"]))
        nxt = pre_w(layers[li + 1]) if li + 1 < len(layers) else None
        if nxt is None:
            (h,) = _tc_node_update(h, P, u, nxt)
        else:
            h, hA, hB = _tc_node_update(h, P, u, nxt)

    Pp = _sc_scatter_add(h, batch, z_graphs)
    W2p = jnp.pad(params["pW2"], ((0, 0), (0, D - params["pW2"].shape[1])))
    b2p = jnp.pad(params["pb2"], (0, D - params["pb2"].shape[0])).reshape(1, D)
    out_full = _tc_head(Pp, params["pW1"], row(params["pb1"]), W2p, b2p)
    return out_full[:, :1]
